# baseline (speedup 1.0000x reference)
import functools

import jax
import jax.numpy as jnp
from jax.experimental import pallas as pl
from jax.experimental.pallas import tpu as pltpu
from jax.experimental.pallas import tpu_sc as plsc

C_IN = 128
M_OUT = 256
ROW_TILE = 16384
_NSUB = 16
_NWORK = 2 * _NSUB
_JPW = C_IN // _NWORK


def _sc_fold_body(oi_hbm, ti_hbm, w_hbm, w2_hbm, oi_v, ti_v, wg_v, acc_v):
    c = jax.lax.axis_index("c")
    s = jax.lax.axis_index("s")
    wid = c * _NSUB + s

    pltpu.sync_copy(oi_hbm, oi_v)
    pltpu.sync_copy(ti_hbm, ti_v)
    pltpu.sync_copy(w_hbm.at[ti_v.at[0]], wg_v)

    lo = wid * _JPW

    @pl.loop(0, _JPW)
    def _(r):
        @pl.loop(0, M_OUT, step=16)
        def _(k):
            acc_v.at[r, pl.ds(k, 16)][...] = jnp.zeros((16,), jnp.float32)

    for ch in range(C_IN // 16):
        idx_vec = oi_v[0, pl.ds(ch * 16, 16)]
        for lane in range(16):
            j = ch * 16 + lane
            t = idx_vec[lane]
            local = t - lo

            @pl.when(jnp.logical_and(t >= lo, t < lo + _JPW))
            def _(j=j, local=local):
                @pl.loop(0, M_OUT, step=16)
                def _(k, j=j, local=local):
                    acc_v.at[local, pl.ds(k, 16)][...] = (
                        acc_v.at[local, pl.ds(k, 16)][...]
                        + wg_v.at[j, pl.ds(k, 16)][...]
                    )

    pltpu.sync_copy(acc_v, w2_hbm.at[pl.ds(lo, _JPW)])


def _mix_kernel(w2_ref, b_ref, x_ref, o_ref):
    o_ref[...] = (
        jnp.dot(x_ref[...], w2_ref[...], preferred_element_type=jnp.float32)
        + b_ref[...]
    )


@functools.partial(jax.jit, static_argnames=())
def kernel(eeg, orig_idx, target_idx, W, b):
    B, T, C = eeg.shape
    M = W.shape[1]
    oi = orig_idx.astype(jnp.int32).reshape(1, C_IN)
    ti = target_idx.astype(jnp.int32).reshape(1, C_IN)

    sc_fold = pl.kernel(
        _sc_fold_body,
        out_type=jax.ShapeDtypeStruct((C_IN, M_OUT), jnp.float32),
        mesh=plsc.VectorSubcoreMesh(core_axis_name="c", subcore_axis_name="s"),
        scratch_types=[
            pltpu.VMEM((1, C_IN), jnp.int32),
            pltpu.VMEM((1, C_IN), jnp.int32),
            pltpu.VMEM((C_IN, M_OUT), jnp.float32),
            pltpu.VMEM((_JPW, M_OUT), jnp.float32),
        ],
    )
    w2 = sc_fold(oi, ti, W)

    x = eeg.reshape(B * T, C)
    rows = B * T
    out = pl.pallas_call(
        _mix_kernel,
        grid=(rows // ROW_TILE,),
        in_specs=[
            pl.BlockSpec((C_IN, M_OUT), lambda i: (0, 0)),
            pl.BlockSpec((1, M_OUT), lambda i: (0, 0)),
            pl.BlockSpec((ROW_TILE, C_IN), lambda i: (i, 0)),
        ],
        out_specs=pl.BlockSpec((ROW_TILE, M_OUT), lambda i: (i, 0)),
        out_shape=jax.ShapeDtypeStruct((rows, M_OUT), jnp.float32),
        compiler_params=pltpu.CompilerParams(
            dimension_semantics=("arbitrary",),
            vmem_limit_bytes=120 * 1024 * 1024,
        ),
    )(w2, b.reshape(1, M_OUT), x)
    return out.reshape(B, T, M)


# speedup vs baseline: 1.1320x; 1.1320x over previous
import functools

import jax
import jax.numpy as jnp
from jax.experimental import pallas as pl
from jax.experimental.pallas import tpu as pltpu
from jax.experimental.pallas import tpu_sc as plsc

C_IN = 128
M_OUT = 256
ROW_TILE = 16384
_NSUB = 16
_JPW = C_IN // _NSUB
_OI_PAD = C_IN + 16


def _sc_fold_body(oi_hbm, ti_hbm, w_hbm, w2_hbm, oi_v, ti_v, wg_v, acc_v):
    c = jax.lax.axis_index("c")
    s = jax.lax.axis_index("s")

    @pl.when(c == 0)
    def _():
        pltpu.sync_copy(oi_hbm, oi_v)
        pltpu.sync_copy(ti_hbm, ti_v)
        pltpu.sync_copy(w_hbm.at[ti_v.at[0]], wg_v)

        lo = s * _JPW

        @pl.loop(0, _JPW)
        def _(r):
            @pl.loop(0, M_OUT, step=16)
            def _(k):
                acc_v.at[r, pl.ds(k, 16)][...] = jnp.zeros((16,), jnp.float32)

        @pl.loop(0, C_IN)
        def _(j):
            t = oi_v[0, pl.ds(j, 16)][0]
            local = t - lo

            @pl.when(jnp.logical_and(t >= lo, t < lo + _JPW))
            def _():
                @pl.loop(0, M_OUT, step=16)
                def _(k):
                    acc_v.at[local, pl.ds(k, 16)][...] = (
                        acc_v.at[local, pl.ds(k, 16)][...]
                        + wg_v.at[j, pl.ds(k, 16)][...]
                    )

        pltpu.sync_copy(acc_v, w2_hbm.at[pl.ds(lo, _JPW)])


def _mix_kernel(w2_ref, b_ref, x_ref, o_ref):
    o_ref[...] = (
        jnp.dot(x_ref[...], w2_ref[...], preferred_element_type=jnp.float32)
        + b_ref[...]
    )


@functools.partial(jax.jit, static_argnames=())
def kernel(eeg, orig_idx, target_idx, W, b):
    B, T, C = eeg.shape
    M = W.shape[1]
    oi = jnp.pad(orig_idx.astype(jnp.int32), (0, _OI_PAD - C_IN),
                 constant_values=-1).reshape(1, _OI_PAD)
    ti = target_idx.astype(jnp.int32).reshape(1, C_IN)

    sc_fold = pl.kernel(
        _sc_fold_body,
        out_type=jax.ShapeDtypeStruct((C_IN, M_OUT), jnp.float32),
        mesh=plsc.VectorSubcoreMesh(core_axis_name="c", subcore_axis_name="s"),
        scratch_types=[
            pltpu.VMEM((1, _OI_PAD), jnp.int32),
            pltpu.VMEM((1, C_IN), jnp.int32),
            pltpu.VMEM((C_IN, M_OUT), jnp.float32),
            pltpu.VMEM((_JPW, M_OUT), jnp.float32),
        ],
    )
    w2 = sc_fold(oi, ti, W)

    x = eeg.reshape(B * T, C)
    rows = B * T
    out = pl.pallas_call(
        _mix_kernel,
        grid=(rows // ROW_TILE,),
        in_specs=[
            pl.BlockSpec((C_IN, M_OUT), lambda i: (0, 0)),
            pl.BlockSpec((1, M_OUT), lambda i: (0, 0)),
            pl.BlockSpec((ROW_TILE, C_IN), lambda i: (i, 0)),
        ],
        out_specs=pl.BlockSpec((ROW_TILE, M_OUT), lambda i: (i, 0)),
        out_shape=jax.ShapeDtypeStruct((rows, M_OUT), jnp.float32),
        compiler_params=pltpu.CompilerParams(
            dimension_semantics=("arbitrary",),
            vmem_limit_bytes=120 * 1024 * 1024,
        ),
    )(w2, b.reshape(1, M_OUT), x)
    return out.reshape(B, T, M)


# speedup vs baseline: 1.1414x; 1.0082x over previous
import functools

import jax
import jax.numpy as jnp
from jax.experimental import pallas as pl
from jax.experimental.pallas import tpu as pltpu
from jax.experimental.pallas import tpu_sc as plsc

C_IN = 128
M_OUT = 256
ROW_TILE = 16384
_NSUB = 16
_JPW = C_IN // _NSUB
_OI_PAD = C_IN + 16


def _sc_fold_body(oi_hbm, ti_hbm, w_hbm, w2_hbm, oi_v, ti_v, wg_v, acc_v,
                  sem_oi, sem_wg):
    c = jax.lax.axis_index("c")
    s = jax.lax.axis_index("s")

    @pl.when(c == 0)
    def _():
        oi_cp = pltpu.async_copy(oi_hbm, oi_v, sem_oi)
        pltpu.sync_copy(ti_hbm, ti_v)
        wg_cp = pltpu.async_copy(w_hbm.at[ti_v.at[0]], wg_v, sem_wg)

        lo = s * _JPW

        @pl.loop(0, _JPW)
        def _(r):
            @pl.loop(0, M_OUT, step=16)
            def _(k):
                acc_v.at[r, pl.ds(k, 16)][...] = jnp.zeros((16,), jnp.float32)

        oi_cp.wait()
        wg_cp.wait()

        @pl.loop(0, C_IN)
        def _(j):
            t = oi_v[0, pl.ds(j, 16)][0]
            local = t - lo

            @pl.when(jnp.logical_and(t >= lo, t < lo + _JPW))
            def _():
                @pl.loop(0, M_OUT, step=16)
                def _(k):
                    acc_v.at[local, pl.ds(k, 16)][...] = (
                        acc_v.at[local, pl.ds(k, 16)][...]
                        + wg_v.at[j, pl.ds(k, 16)][...]
                    )

        pltpu.sync_copy(acc_v, w2_hbm.at[pl.ds(lo, _JPW)])


def _mix_kernel(w2_ref, b_ref, x_ref, o_ref):
    o_ref[...] = (
        jnp.dot(x_ref[...], w2_ref[...], preferred_element_type=jnp.float32)
        + b_ref[...]
    )


@functools.partial(jax.jit, static_argnames=())
def kernel(eeg, orig_idx, target_idx, W, b):
    B, T, C = eeg.shape
    M = W.shape[1]
    oi = jnp.pad(orig_idx.astype(jnp.int32), (0, _OI_PAD - C_IN),
                 constant_values=-1).reshape(1, _OI_PAD)
    ti = target_idx.astype(jnp.int32).reshape(1, C_IN)

    sc_fold = pl.kernel(
        _sc_fold_body,
        out_type=jax.ShapeDtypeStruct((C_IN, M_OUT), jnp.float32),
        mesh=plsc.VectorSubcoreMesh(core_axis_name="c", subcore_axis_name="s"),
        scratch_types=[
            pltpu.VMEM((1, _OI_PAD), jnp.int32),
            pltpu.VMEM((1, C_IN), jnp.int32),
            pltpu.VMEM((C_IN, M_OUT), jnp.float32),
            pltpu.VMEM((_JPW, M_OUT), jnp.float32),
            pltpu.SemaphoreType.DMA,
            pltpu.SemaphoreType.DMA,
        ],
    )
    w2 = sc_fold(oi, ti, W)

    x = eeg.reshape(B * T, C)
    rows = B * T
    out = pl.pallas_call(
        _mix_kernel,
        grid=(rows // ROW_TILE,),
        in_specs=[
            pl.BlockSpec((C_IN, M_OUT), lambda i: (0, 0)),
            pl.BlockSpec((1, M_OUT), lambda i: (0, 0)),
            pl.BlockSpec((ROW_TILE, C_IN), lambda i: (i, 0)),
        ],
        out_specs=pl.BlockSpec((ROW_TILE, M_OUT), lambda i: (i, 0)),
        out_shape=jax.ShapeDtypeStruct((rows, M_OUT), jnp.float32),
        compiler_params=pltpu.CompilerParams(
            dimension_semantics=("arbitrary",),
            vmem_limit_bytes=120 * 1024 * 1024,
        ),
    )(w2, b.reshape(1, M_OUT), x)
    return out.reshape(B, T, M)


# speedup vs baseline: 1.1587x; 1.0152x over previous
import functools

import jax
import jax.numpy as jnp
from jax.experimental import pallas as pl
from jax.experimental.pallas import tpu as pltpu
from jax.experimental.pallas import tpu_sc as plsc

C_IN = 128
M_OUT = 256
ROW_TILE = 16384
_NSUB = 16
_JPW = C_IN // _NSUB
_OI_PAD = C_IN + 16


def _sc_fold_body(oi_hbm, ti_hbm, w_hbm, w2_hbm, oi_v, ti_v, wg_v, acc_v,
                  sem_oi, sem_wg):
    c = jax.lax.axis_index("c")
    s = jax.lax.axis_index("s")

    @pl.when(c == 0)
    def _():
        oi_cp = pltpu.async_copy(oi_hbm, oi_v, sem_oi)
        pltpu.sync_copy(ti_hbm, ti_v)
        wg_cp = pltpu.async_copy(w_hbm.at[ti_v.at[0]], wg_v, sem_wg)

        lo = s * _JPW

        @pl.loop(0, _JPW)
        def _(r):
            @pl.loop(0, M_OUT, step=16)
            def _(k):
                acc_v.at[r, pl.ds(k, 16)][...] = jnp.zeros((16,), jnp.float32)

        oi_cp.wait()
        wg_cp.wait()

        @pl.loop(0, C_IN)
        def _(j):
            t = oi_v[0, pl.ds(j, 16)][0]
            local = t - lo

            @pl.when(jnp.logical_and(t >= lo, t < lo + _JPW))
            def _():
                @pl.loop(0, M_OUT, step=16)
                def _(k):
                    acc_v.at[local, pl.ds(k, 16)][...] = (
                        acc_v.at[local, pl.ds(k, 16)][...]
                        + wg_v.at[j, pl.ds(k, 16)][...]
                    )

        pltpu.sync_copy(acc_v, w2_hbm.at[pl.ds(lo, _JPW)])


def _mix_kernel(w2_ref, b_ref, x_ref, o_ref):
    o_ref[...] = (
        jnp.dot(x_ref[...], w2_ref[...], preferred_element_type=jnp.float32)
        + b_ref[...]
    )


@functools.partial(jax.jit, static_argnames=())
def kernel(eeg, orig_idx, target_idx, W, b):
    B, T, C = eeg.shape
    M = W.shape[1]
    oi = jnp.pad(orig_idx.astype(jnp.int32), (0, _OI_PAD - C_IN),
                 constant_values=-1).reshape(1, _OI_PAD)
    ti = target_idx.astype(jnp.int32).reshape(1, C_IN)

    sc_fold = pl.kernel(
        _sc_fold_body,
        out_type=jax.ShapeDtypeStruct((C_IN, M_OUT), jnp.float32),
        mesh=plsc.VectorSubcoreMesh(core_axis_name="c", subcore_axis_name="s",
                                    num_cores=1),
        scratch_types=[
            pltpu.VMEM((1, _OI_PAD), jnp.int32),
            pltpu.VMEM((1, C_IN), jnp.int32),
            pltpu.VMEM((C_IN, M_OUT), jnp.float32),
            pltpu.VMEM((_JPW, M_OUT), jnp.float32),
            pltpu.SemaphoreType.DMA,
            pltpu.SemaphoreType.DMA,
        ],
    )
    w2 = sc_fold(oi, ti, W)

    x = eeg.reshape(B * T, C)
    rows = B * T
    out = pl.pallas_call(
        _mix_kernel,
        grid=(rows // ROW_TILE,),
        in_specs=[
            pl.BlockSpec((C_IN, M_OUT), lambda i: (0, 0)),
            pl.BlockSpec((1, M_OUT), lambda i: (0, 0)),
            pl.BlockSpec((ROW_TILE, C_IN), lambda i: (i, 0)),
        ],
        out_specs=pl.BlockSpec((ROW_TILE, M_OUT), lambda i: (i, 0)),
        out_shape=jax.ShapeDtypeStruct((rows, M_OUT), jnp.float32),
        compiler_params=pltpu.CompilerParams(
            dimension_semantics=("arbitrary",),
            vmem_limit_bytes=120 * 1024 * 1024,
        ),
    )(w2, b.reshape(1, M_OUT), x)
    return out.reshape(B, T, M)
